# baseline (speedup 1.0000x reference)
import functools

import jax
import jax.numpy as jnp
from jax import lax
from jax.experimental import pallas as pl
from jax.experimental.pallas import tpu as pltpu
from jax.experimental.pallas import tpu_sc as plsc

F32 = jnp.float32
D = 128
BLK = 2000


def _pick_ch(n, E, nbuf):
    for ch in (128, 64, 32):
        if E % ch == 0 and 16 * (nbuf * ch * D + 4 * ch) + n * D <= 2_060_000:
            return ch
    return 16

_pallas_call = pl.pallas_call


@functools.lru_cache(None)
def _mm_bias(N, K, ln):
    grid = N // BLK

    def kern(*refs):
        if ln:
            x_ref, w_ref, b_ref, g_ref, gb_ref, o_ref = refs
        else:
            x_ref, w_ref, b_ref, o_ref = refs
        acc = jnp.dot(x_ref[...], w_ref[...], preferred_element_type=F32)
        acc = acc + b_ref[...]
        if ln:
            mu = jnp.mean(acc, axis=-1, keepdims=True)
            var = jnp.mean((acc - mu) ** 2, axis=-1, keepdims=True)
            acc = (acc - mu) / jnp.sqrt(var + 1e-5) * g_ref[...] + gb_ref[...]
        o_ref[...] = acc

    in_specs = [
        pl.BlockSpec((BLK, K), lambda i: (i, 0)),
        pl.BlockSpec((K, D), lambda i: (0, 0)),
        pl.BlockSpec((1, D), lambda i: (0, 0)),
    ]
    if ln:
        in_specs += [pl.BlockSpec((1, D), lambda i: (0, 0)),
                     pl.BlockSpec((1, D), lambda i: (0, 0))]
    return _pallas_call(
        kern, grid=(grid,), in_specs=in_specs,
        out_specs=pl.BlockSpec((BLK, D), lambda i: (i, 0)),
        out_shape=jax.ShapeDtypeStruct((N, D), F32))


def _lin_tc(p, x, ln=None):
    N, K = x.shape
    wt = jnp.transpose(p["W"])
    b = p["b"].reshape(1, D)
    if ln is None:
        return _mm_bias(N, K, False)(x, wt, b)
    return _mm_bias(N, K, True)(x, wt, b, ln["g"].reshape(1, D),
                                ln["b"].reshape(1, D))


@functools.lru_cache(None)
def _nprep(N, with_c2p):
    grid = N // BLK

    def kern(*refs):
        if with_c2p:
            (h_ref, p_ref, wb1, bb1, wb2, bb2, wa2h, wa2p, ba2, wc2, bc2,
             o1, o2, o3, o4) = refs
        else:
            h_ref, p_ref, wb1, bb1, wb2, bb2, wa2h, wa2p, ba2, o1, o2, o3 = refs
        h = h_ref[...]
        p = p_ref[...]
        o1[...] = jnp.dot(h, wb1[...], preferred_element_type=F32) + bb1[...]
        o2[...] = jnp.dot(h, wb2[...], preferred_element_type=F32) + bb2[...]
        o3[...] = (jnp.dot(h, wa2h[...], preferred_element_type=F32)
                   + jnp.dot(p, wa2p[...], preferred_element_type=F32)
                   + ba2[...])
        if with_c2p:
            o4[...] = jnp.dot(p, wc2[...], preferred_element_type=F32) + bc2[...]

    mat = pl.BlockSpec((D, D), lambda i: (0, 0))
    vec = pl.BlockSpec((1, D), lambda i: (0, 0))
    row = pl.BlockSpec((BLK, D), lambda i: (i, 0))
    if with_c2p:
        in_specs = [row, row, mat, vec, mat, vec, mat, mat, vec, mat, vec]
        n_out = 4
    else:
        in_specs = [row, row, mat, vec, mat, vec, mat, mat, vec]
        n_out = 3
    return _pallas_call(
        kern, grid=(grid,), in_specs=in_specs,
        out_specs=[row] * n_out,
        out_shape=[jax.ShapeDtypeStruct((N, D), F32)] * n_out)


@functools.lru_cache(None)
def _nfin(N, with_p):
    def kern(*refs):
        if with_p:
            (h_ref, p_ref, ha_ref, pa_ref, s0_ref, s1_ref,
             wa1h, wa1p, ba1, wc1, bc1, g_ref, gb_ref, ho_ref, po_ref) = refs
        else:
            (h_ref, p_ref, ha_ref, hb_ref, s0_ref, s1_ref,
             wa1h, wa1p, ba1, g_ref, gb_ref, ho_ref) = refs
        h = h_ref[...]
        p = p_ref[...]
        inv = 1.0 / (s0_ref[...] + s1_ref[...] + 1e-6)
        if with_p:
            h_agg = ha_ref[...] * inv
        else:
            h_agg = (ha_ref[...] + hb_ref[...]) * inv
        h_new = (jnp.dot(h, wa1h[...], preferred_element_type=F32)
                 + jnp.dot(p, wa1p[...], preferred_element_type=F32)
                 + ba1[...] + h_agg)
        mu = jnp.mean(h_new, axis=0, keepdims=True)
        var = jnp.mean((h_new - mu) ** 2, axis=0, keepdims=True)
        hb = (h_new - mu) / jnp.sqrt(var + 1e-5) * g_ref[...] + gb_ref[...]
        ho_ref[...] = h + jnp.maximum(hb, 0.0)
        if with_p:
            p_new = (jnp.dot(p, wc1[...], preferred_element_type=F32)
                     + bc1[...] + pa_ref[...] * inv)
            po_ref[...] = p + jnp.tanh(p_new)

    n_out = 2 if with_p else 1
    return _pallas_call(
        kern,
        out_shape=[jax.ShapeDtypeStruct((N, D), F32)] * n_out)


@functools.lru_cache(None)
def _estats(E):
    grid = E // BLK

    def kern(x_ref, o_ref):
        @pl.when(pl.program_id(0) == 0)
        def _():
            o_ref[...] = jnp.zeros((8, D), F32)
        x = x_ref[...]
        o_ref[0:1, :] += jnp.sum(x, axis=0, keepdims=True)
        o_ref[1:2, :] += jnp.sum(x * x, axis=0, keepdims=True)

    return _pallas_call(
        kern, grid=(grid,),
        in_specs=[pl.BlockSpec((BLK, D), lambda i: (i, 0))],
        out_specs=pl.BlockSpec((8, D), lambda i: (0, 0)),
        out_shape=jax.ShapeDtypeStruct((8, D), F32))


@functools.lru_cache(None)
def _eprep(E):
    grid = E // BLK
    inv = 1.0 / E

    def kern(e_ref, hat_ref, st_ref, wb3, bb3, g_ref, gb_ref, eo_ref, b3_ref):
        mu = st_ref[0:1, :] * inv
        var = st_ref[1:2, :] * inv - mu * mu
        hb = (hat_ref[...] - mu) / jnp.sqrt(var + 1e-5) * g_ref[...] + gb_ref[...]
        eo = e_ref[...] + jnp.maximum(hb, 0.0)
        eo_ref[...] = eo
        b3_ref[...] = jnp.dot(eo, wb3[...], preferred_element_type=F32) + bb3[...]

    row = pl.BlockSpec((BLK, D), lambda i: (i, 0))
    return _pallas_call(
        kern, grid=(grid,),
        in_specs=[row, row,
                  pl.BlockSpec((8, D), lambda i: (0, 0)),
                  pl.BlockSpec((D, D), lambda i: (0, 0)),
                  pl.BlockSpec((1, D), lambda i: (0, 0)),
                  pl.BlockSpec((1, D), lambda i: (0, 0)),
                  pl.BlockSpec((1, D), lambda i: (0, 0))],
        out_specs=[row, row],
        out_shape=[jax.ShapeDtypeStruct((E, D), F32)] * 2)


_MESH = dict(core_axis_name="c", subcore_axis_name="s")
_NC, _NS = 2, 16


def _zero_rows(r1):
    for j in range(8):
        for k8 in range(D // 16):
            r1[j, pl.ds(k8 * 16, 16)] = jnp.zeros((16,), F32)


def _acc_init(r1, acc, s, n):
    _zero_rows(r1)
    nz = n // 8
    cnt = (nz - s + _NS - 1) // _NS

    def zbody(k, _):
        r0 = (s + k * _NS) * 8
        pltpu.sync_copy(r1.at[pl.ds(0, 8)], acc.at[pl.ds(r0, 8)])
        return 0

    lax.fori_loop(0, cnt, zbody, 0)


def _acc_out(acc, out_ref, s, n):
    nz = n // 8
    cnt = (nz - s + _NS - 1) // _NS

    def obody(k, _):
        r0 = (s + k * _NS) * 8
        pltpu.sync_copy(acc.at[pl.ds(r0, 8)], out_ref.at[pl.ds(r0, 8)])
        return 0

    lax.fori_loop(0, cnt, obody, 0)


@functools.lru_cache(None)
def _sc_edge_pass(n, E, mode):
    nbuf = 6 if mode == "pass1" else 4
    CH = _pick_ch(n, E, nbuf)
    n_chunks = E // CH
    stride = _NS if mode == "pass2_hp" else _NC * _NS
    mesh = plsc.VectorSubcoreMesh(**_MESH)

    n_out = 3 if mode == "pass1" else 2
    out_type = tuple([jax.ShapeDtypeStruct((E, D), F32)] * (n_out - 2)
                     + [jax.ShapeDtypeStruct((n, D), F32)] * 2)
    scratch = ([pltpu.VMEM((CH,), jnp.int32)] * 4
               + [pltpu.VMEM((CH, D), F32)] * nbuf
               + [pltpu.VMEM_SHARED((n, D), F32)]
               + [pltpu.SemaphoreType.DMA] * 2)

    @functools.partial(pl.kernel, mesh=mesh, out_type=out_type,
                       scratch_types=scratch)
    def body(*refs):
        if mode == "pass1":
            (src_h, dst_h, b1_h, b2_h, b3_h, hat_h, outa_h, outb_h,
             sv0, sv1, dv0, dv1, r10, r11, r20, r21, r30, r31,
             acc, sg0, sg1) = refs
            r1 = (r10, r11)
            r2 = (r20, r21)
            r3 = (r30, r31)
        else:
            (src_h, dst_h, hat_h, tbl_h, outa_h, outb_h,
             sv0, sv1, dv0, dv1, r20, r21, r30, r31,
             acc, sg0, sg1) = refs
            r1 = (r20, r21)
            r2 = (r20, r21)
            r3 = (r30, r31)
        srcv = (sv0, sv1)
        dstv = (dv0, dv1)
        s_g = (sg0, sg1)

        c = lax.axis_index("c")
        s = lax.axis_index("s")
        wid = s if mode == "pass2_hp" else s * _NC + c

        _acc_init(r3[0], acc, s, n)
        plsc.subcore_barrier()

        cnt = (n_chunks - wid + stride - 1) // stride

        def base_of(k):
            return jnp.minimum(wid + k * stride, n_chunks - 1) * CH

        def load_idx(b, k):
            base = base_of(k)
            pltpu.sync_copy(src_h.at[pl.ds(base, CH)], srcv[b])
            pltpu.sync_copy(dst_h.at[pl.ds(base, CH)], dstv[b])
            if mode == "pass2_hp":
                off = c * n
                for j in range(CH // 16):
                    sl = pl.ds(j * 16, 16)
                    srcv[b][sl] = srcv[b][sl] + off

        def issue_gathers(b, k):
            base = base_of(k)
            if mode == "pass1":
                return (pltpu.async_copy(b1_h.at[srcv[b]], r1[b], s_g[b]),
                        pltpu.async_copy(b2_h.at[dstv[b]], r2[b], s_g[b]),
                        pltpu.async_copy(b3_h.at[pl.ds(base, CH)], r3[b], s_g[b]))
            return (pltpu.async_copy(tbl_h.at[srcv[b]], r2[b], s_g[b]),
                    pltpu.async_copy(hat_h.at[pl.ds(base, CH)], r3[b], s_g[b]))

        def compute(b):
            if mode == "pass1":
                def ebody(e, _):
                    for j in range(D // 16):
                        sl = pl.ds(j * 16, 16)
                        hat = r1[b][e, sl] + r2[b][e, sl] + r3[b][e, sl]
                        r3[b][e, sl] = hat
                        r1[b][e, sl] = 1.0 / (1.0 + jnp.exp(-hat))
                    return 0
            else:
                def ebody(e, _):
                    for j in range(D // 16):
                        sl = pl.ds(j * 16, 16)
                        t = 1.0 + jnp.exp(-r3[b][e, sl])
                        r2[b][e, sl] = r2[b][e, sl] / t
                    return 0
            lax.fori_loop(0, CH, ebody, 0, unroll=4)

        def sync_writes(b, k):
            base = base_of(k)
            if mode == "pass1":
                pltpu.sync_copy(r3[b], hat_h.at[pl.ds(base, CH)])
                pltpu.sync_copy(r1[b], acc.at[dstv[b]], add=True)
            else:
                pltpu.sync_copy(r2[b], acc.at[dstv[b]], add=True)

        def stage(b, k):
            nb = 1 - b
            load_idx(nb, k + 1)
            cps = issue_gathers(nb, k + 1)
            compute(b)
            sync_writes(b, k)
            for cp in cps:
                cp.wait()

        load_idx(0, 0)
        for cp in issue_gathers(0, 0):
            cp.wait()

        def gbody(g, _):
            stage(0, g * 2)
            stage(1, g * 2 + 1)
            return 0

        lax.fori_loop(0, cnt // 2, gbody, 0)

        @pl.when(cnt % 2 == 1)
        def _():
            stage(0, cnt - 1)

        plsc.subcore_barrier()

        @pl.when(c == 0)
        def _():
            _acc_out(acc, outa_h, s, n)

        @pl.when(c == 1)
        def _():
            _acc_out(acc, outb_h, s, n)

    return body


def _sc_pass1(n, E):
    return _sc_edge_pass(n, E, "pass1")


def _sc_pass2_hp(n, E):
    return _sc_edge_pass(n, E, "pass2_hp")


def _sc_pass2_h(n, E):
    return _sc_edge_pass(n, E, "pass2_h")


def _split_w(w):
    return jnp.transpose(w[:, :D]), jnp.transpose(w[:, D:])


def _gated_layer(lp, src, dst, n, h, p, b3e, with_p):
    wb1 = jnp.transpose(lp["B1"]["W"])
    wb2 = jnp.transpose(lp["B2"]["W"])
    wa2h, wa2p = _split_w(lp["A2"]["W"])
    E = src.shape[0]
    if with_p:
        wc2 = jnp.transpose(lp["C2"]["W"])
        b1h, b2h, vij, c2p = _nprep(n, True)(
            h, p, wb1, lp["B1"]["b"].reshape(1, D), wb2,
            lp["B2"]["b"].reshape(1, D), wa2h, wa2p,
            lp["A2"]["b"].reshape(1, D), wc2, lp["C2"]["b"].reshape(1, D))
    else:
        b1h, b2h, vij = _nprep(n, False)(
            h, p, wb1, lp["B1"]["b"].reshape(1, D), wb2,
            lp["B2"]["b"].reshape(1, D), wa2h, wa2p,
            lp["A2"]["b"].reshape(1, D))

    hat, ss0, ss1 = _sc_pass1(n, E)(src, dst, b1h, b2h, b3e)

    wa1h, wa1p = _split_w(lp["A1"]["W"])
    if with_p:
        h_agg, p_agg = _sc_pass2_hp(n, E)(
            src, dst, hat, jnp.concatenate([vij, c2p], axis=0))
        h_out, p_out = _nfin(n, True)(
            h, p, h_agg, p_agg, ss0, ss1,
            wa1h, wa1p, lp["A1"]["b"].reshape(1, D),
            jnp.transpose(lp["C1"]["W"]), lp["C1"]["b"].reshape(1, D),
            lp["bn_h_g"].reshape(1, D), lp["bn_h_b"].reshape(1, D))
        return h_out, p_out, hat
    ha0, ha1 = _sc_pass2_h(n, E)(src, dst, hat, vij)
    (h_out,) = _nfin(n, False)(
        h, p, ha0, ha1, ss0, ss1, wa1h, wa1p, lp["A1"]["b"].reshape(1, D),
        lp["bn_h_g"].reshape(1, D), lp["bn_h_b"].reshape(1, D))
    return h_out, None, hat


def _edge_in(lp, E, e_prev, hat_prev, st_prev):
    wb3 = jnp.transpose(lp["B3"]["W"])
    bb3 = lp["B3"]["b"].reshape(1, D)
    if hat_prev is None:
        b3e = _mm_bias(E, D, False)(e_prev, wb3, bb3)
        return e_prev, b3e
    e_out, b3e = _eprep(E)(e_prev, hat_prev, st_prev, wb3, bb3,
                           lp["bn_e_g"].reshape(1, D), lp["bn_e_b"].reshape(1, D))
    return e_out, b3e


def kernel(xp, ep_feats, pp_pos, edge_index_p, xl, el_feats, pl_pos,
           edge_index_l, ec_feats, edge_index_c, params):
    enc = params["enc"]
    n_p = xp.shape[0]
    n_l = xl.shape[0]
    n_c = n_p + n_l

    hp = _lin_tc(enc["p_node"], xp, ln=params["ln_p"])
    hl = _lin_tc(enc["l_node"], xl, ln=params["ln_l"])
    pp = _lin_tc(enc["p_pose"], pp_pos)
    plv = _lin_tc(enc["l_pose"], pl_pos)
    ep = _lin_tc(enc["p_edge"], ep_feats)
    el = _lin_tc(enc["l_edge"], el_feats)
    ec = _lin_tc(enc["c_edge"], ec_feats)

    sp, dp = edge_index_p[0], edge_index_p[1]
    sl, dl = edge_index_l[0], edge_index_l[1]
    sc, dc = edge_index_c[0], edge_index_c[1]
    E_p, E_l, E_c = sp.shape[0], sl.shape[0], sc.shape[0]

    hat_p = hat_l = hat_c = None
    st_p = st_l = st_c = None
    hc = None

    for i in range(3):
        lp_p = params["prot"][i]
        lp_l = params["lig"][i]
        lp_c = params["comp"][i]

        ep, b3e_p = _edge_in(lp_p, E_p, ep, hat_p, st_p)
        hp, pp, hat_p = _gated_layer(lp_p, sp, dp, n_p, hp, pp, b3e_p, True)

        el, b3e_l = _edge_in(lp_l, E_l, el, hat_l, st_l)
        hl, plv, hat_l = _gated_layer(lp_l, sl, dl, n_l, hl, plv, b3e_l, True)

        hc_in = jnp.concatenate([hp, hl], axis=0)
        pc_in = jnp.concatenate([pp, plv], axis=0)
        ec, b3e_c = _edge_in(lp_c, E_c, ec, hat_c, st_c)
        hc, _, hat_c = _gated_layer(lp_c, sc, dc, n_c, hc_in, pc_in,
                                    b3e_c, False)
        hp = hc[:n_p]
        hl = hc[n_p:]

        if i < 2:
            st_p = _estats(E_p)(hat_p)
            st_l = _estats(E_l)(hat_l)
            st_c = _estats(E_c)(hat_c)

    return (hp, hl, hc)


# speedup vs baseline: 2.9547x; 2.9547x over previous
import functools

import jax
import jax.numpy as jnp
from jax import lax
from jax.experimental import pallas as pl
from jax.experimental.pallas import tpu as pltpu
from jax.experimental.pallas import tpu_sc as plsc

F32 = jnp.float32
D = 128
BLK = 2000


def _pick_ch(n, E, nbuf):
    for ch in (128, 64, 32):
        if E % ch == 0 and 16 * (nbuf * ch * D + 4 * ch) + n * D <= 2_060_000:
            return ch
    return 16

_pallas_call = pl.pallas_call


@functools.lru_cache(None)
def _mm_bias(N, K, ln):
    grid = N // BLK

    def kern(*refs):
        if ln:
            x_ref, w_ref, b_ref, g_ref, gb_ref, o_ref = refs
        else:
            x_ref, w_ref, b_ref, o_ref = refs
        acc = jnp.dot(x_ref[...], w_ref[...], preferred_element_type=F32)
        acc = acc + b_ref[...]
        if ln:
            mu = jnp.mean(acc, axis=-1, keepdims=True)
            var = jnp.mean((acc - mu) ** 2, axis=-1, keepdims=True)
            acc = (acc - mu) / jnp.sqrt(var + 1e-5) * g_ref[...] + gb_ref[...]
        o_ref[...] = acc

    in_specs = [
        pl.BlockSpec((BLK, K), lambda i: (i, 0)),
        pl.BlockSpec((K, D), lambda i: (0, 0)),
        pl.BlockSpec((1, D), lambda i: (0, 0)),
    ]
    if ln:
        in_specs += [pl.BlockSpec((1, D), lambda i: (0, 0)),
                     pl.BlockSpec((1, D), lambda i: (0, 0))]
    return _pallas_call(
        kern, grid=(grid,), in_specs=in_specs,
        out_specs=pl.BlockSpec((BLK, D), lambda i: (i, 0)),
        out_shape=jax.ShapeDtypeStruct((N, D), F32))


def _lin_tc(p, x, ln=None):
    N, K = x.shape
    wt = jnp.transpose(p["W"])
    b = p["b"].reshape(1, D)
    if ln is None:
        return _mm_bias(N, K, False)(x, wt, b)
    return _mm_bias(N, K, True)(x, wt, b, ln["g"].reshape(1, D),
                                ln["b"].reshape(1, D))


@functools.lru_cache(None)
def _nprep(N, with_c2p):
    grid = N // BLK

    def kern(*refs):
        if with_c2p:
            (h_ref, p_ref, wb1, bb1, wb2, bb2, wa2h, wa2p, ba2, wc2, bc2,
             o1, o2, o3, o4) = refs
        else:
            h_ref, p_ref, wb1, bb1, wb2, bb2, wa2h, wa2p, ba2, o1, o2, o3 = refs
        h = h_ref[...]
        p = p_ref[...]
        o1[...] = jnp.dot(h, wb1[...], preferred_element_type=F32) + bb1[...]
        o2[...] = jnp.dot(h, wb2[...], preferred_element_type=F32) + bb2[...]
        o3[...] = (jnp.dot(h, wa2h[...], preferred_element_type=F32)
                   + jnp.dot(p, wa2p[...], preferred_element_type=F32)
                   + ba2[...])
        if with_c2p:
            o4[...] = jnp.dot(p, wc2[...], preferred_element_type=F32) + bc2[...]

    mat = pl.BlockSpec((D, D), lambda i: (0, 0))
    vec = pl.BlockSpec((1, D), lambda i: (0, 0))
    row = pl.BlockSpec((BLK, D), lambda i: (i, 0))
    if with_c2p:
        in_specs = [row, row, mat, vec, mat, vec, mat, mat, vec, mat, vec]
        n_out = 4
    else:
        in_specs = [row, row, mat, vec, mat, vec, mat, mat, vec]
        n_out = 3
    return _pallas_call(
        kern, grid=(grid,), in_specs=in_specs,
        out_specs=[row] * n_out,
        out_shape=[jax.ShapeDtypeStruct((N, D), F32)] * n_out)


@functools.lru_cache(None)
def _nfin(N, with_p):
    def kern(*refs):
        if with_p:
            (h_ref, p_ref, ha_ref, pa_ref, s0_ref, s1_ref,
             wa1h, wa1p, ba1, wc1, bc1, g_ref, gb_ref, ho_ref, po_ref) = refs
        else:
            (h_ref, p_ref, ha_ref, hb_ref, s0_ref, s1_ref,
             wa1h, wa1p, ba1, g_ref, gb_ref, ho_ref) = refs
        h = h_ref[...]
        p = p_ref[...]
        inv = 1.0 / (s0_ref[...] + s1_ref[...] + 1e-6)
        if with_p:
            h_agg = ha_ref[...] * inv
        else:
            h_agg = (ha_ref[...] + hb_ref[...]) * inv
        h_new = (jnp.dot(h, wa1h[...], preferred_element_type=F32)
                 + jnp.dot(p, wa1p[...], preferred_element_type=F32)
                 + ba1[...] + h_agg)
        mu = jnp.mean(h_new, axis=0, keepdims=True)
        var = jnp.mean((h_new - mu) ** 2, axis=0, keepdims=True)
        hb = (h_new - mu) / jnp.sqrt(var + 1e-5) * g_ref[...] + gb_ref[...]
        ho_ref[...] = h + jnp.maximum(hb, 0.0)
        if with_p:
            p_new = (jnp.dot(p, wc1[...], preferred_element_type=F32)
                     + bc1[...] + pa_ref[...] * inv)
            po_ref[...] = p + jnp.tanh(p_new)

    n_out = 2 if with_p else 1
    return _pallas_call(
        kern,
        out_shape=[jax.ShapeDtypeStruct((N, D), F32)] * n_out)


@functools.lru_cache(None)
def _estats(E):
    grid = E // BLK

    def kern(x_ref, o_ref):
        @pl.when(pl.program_id(0) == 0)
        def _():
            o_ref[...] = jnp.zeros((8, D), F32)
        x = x_ref[...]
        o_ref[0:1, :] += jnp.sum(x, axis=0, keepdims=True)
        o_ref[1:2, :] += jnp.sum(x * x, axis=0, keepdims=True)

    return _pallas_call(
        kern, grid=(grid,),
        in_specs=[pl.BlockSpec((BLK, D), lambda i: (i, 0))],
        out_specs=pl.BlockSpec((8, D), lambda i: (0, 0)),
        out_shape=jax.ShapeDtypeStruct((8, D), F32))


@functools.lru_cache(None)
def _eprep(E):
    grid = E // BLK
    inv = 1.0 / E

    def kern(e_ref, hat_ref, st_ref, wb3, bb3, g_ref, gb_ref, eo_ref, b3_ref):
        mu = st_ref[0:1, :] * inv
        var = st_ref[1:2, :] * inv - mu * mu
        hb = (hat_ref[...] - mu) / jnp.sqrt(var + 1e-5) * g_ref[...] + gb_ref[...]
        eo = e_ref[...] + jnp.maximum(hb, 0.0)
        eo_ref[...] = eo
        b3_ref[...] = jnp.dot(eo, wb3[...], preferred_element_type=F32) + bb3[...]

    row = pl.BlockSpec((BLK, D), lambda i: (i, 0))
    return _pallas_call(
        kern, grid=(grid,),
        in_specs=[row, row,
                  pl.BlockSpec((8, D), lambda i: (0, 0)),
                  pl.BlockSpec((D, D), lambda i: (0, 0)),
                  pl.BlockSpec((1, D), lambda i: (0, 0)),
                  pl.BlockSpec((1, D), lambda i: (0, 0)),
                  pl.BlockSpec((1, D), lambda i: (0, 0))],
        out_specs=[row, row],
        out_shape=[jax.ShapeDtypeStruct((E, D), F32)] * 2)


_MESH = dict(core_axis_name="c", subcore_axis_name="s")
_NC, _NS = 2, 16


def _zero_rows(r1):
    for j in range(8):
        for k8 in range(D // 16):
            r1[j, pl.ds(k8 * 16, 16)] = jnp.zeros((16,), F32)


def _acc_init(r1, acc, s, n):
    _zero_rows(r1)
    nz = n // 8
    cnt = (nz - s + _NS - 1) // _NS

    def zbody(k, _):
        r0 = (s + k * _NS) * 8
        pltpu.sync_copy(r1.at[pl.ds(0, 8)], acc.at[pl.ds(r0, 8)])
        return 0

    lax.fori_loop(0, cnt, zbody, 0)


def _acc_out(acc, out_ref, s, n):
    nz = n // 8
    cnt = (nz - s + _NS - 1) // _NS

    def obody(k, _):
        r0 = (s + k * _NS) * 8
        pltpu.sync_copy(acc.at[pl.ds(r0, 8)], out_ref.at[pl.ds(r0, 8)])
        return 0

    lax.fori_loop(0, cnt, obody, 0)


@functools.lru_cache(None)
def _sc_edge_pass(n, E, mode):
    nbuf = 6 if mode == "pass1" else 4
    CH = _pick_ch(n, E, nbuf)
    n_chunks = E // CH
    stride = _NS if mode == "pass2_hp" else _NC * _NS
    mesh = plsc.VectorSubcoreMesh(**_MESH)

    n_out = 3 if mode == "pass1" else 2
    out_type = tuple([jax.ShapeDtypeStruct((E, D), F32)] * (n_out - 2)
                     + [jax.ShapeDtypeStruct((n, D), F32)] * 2)
    scratch = ([pltpu.VMEM((CH,), jnp.int32)] * 4
               + [pltpu.VMEM((CH, D), F32)] * nbuf
               + [pltpu.VMEM_SHARED((n, D), F32)]
               + [pltpu.SemaphoreType.DMA] * 2)

    @functools.partial(pl.kernel, mesh=mesh, out_type=out_type,
                       scratch_types=scratch)
    def body(*refs):
        if mode == "pass1":
            (src_h, dst_h, b1_h, b2_h, b3_h, hat_h, outa_h, outb_h,
             sv0, sv1, dv0, dv1, r10, r11, r20, r21, r30, r31,
             acc, sg0, sg1) = refs
            r1 = (r10, r11)
            r2 = (r20, r21)
            r3 = (r30, r31)
        else:
            (src_h, dst_h, hat_h, tbl_h, outa_h, outb_h,
             sv0, sv1, dv0, dv1, r20, r21, r30, r31,
             acc, sg0, sg1) = refs
            r1 = (r20, r21)
            r2 = (r20, r21)
            r3 = (r30, r31)
        srcv = (sv0, sv1)
        dstv = (dv0, dv1)
        s_g = (sg0, sg1)

        c = lax.axis_index("c")
        s = lax.axis_index("s")
        wid = s if mode == "pass2_hp" else s * _NC + c

        _acc_init(r3[0], acc, s, n)
        plsc.subcore_barrier()

        cnt = (n_chunks - wid + stride - 1) // stride

        def base_of(k):
            return jnp.minimum(wid + k * stride, n_chunks - 1) * CH

        def load_idx(b, k):
            base = base_of(k)
            pltpu.sync_copy(src_h.at[pl.ds(base, CH)], srcv[b])
            pltpu.sync_copy(dst_h.at[pl.ds(base, CH)], dstv[b])
            if mode == "pass2_hp":
                off = c * n
                for j in range(CH // 16):
                    sl = pl.ds(j * 16, 16)
                    srcv[b][sl] = srcv[b][sl] + off

        def issue_gathers(b, k):
            base = base_of(k)
            if mode == "pass1":
                return (pltpu.async_copy(b1_h.at[srcv[b]], r1[b], s_g[b]),
                        pltpu.async_copy(b2_h.at[dstv[b]], r2[b], s_g[b]),
                        pltpu.async_copy(b3_h.at[pl.ds(base, CH)], r3[b], s_g[b]))
            return (pltpu.async_copy(tbl_h.at[srcv[b]], r2[b], s_g[b]),
                    pltpu.async_copy(hat_h.at[pl.ds(base, CH)], r3[b], s_g[b]))

        def compute(b):
            if mode == "pass1":
                def ebody(e, _):
                    for j in range(D // 16):
                        sl = pl.ds(j * 16, 16)
                        hat = r1[b][e, sl] + r2[b][e, sl] + r3[b][e, sl]
                        r3[b][e, sl] = hat
                        r1[b][e, sl] = 1.0 / (1.0 + jnp.exp(-hat))
                    return 0
            else:
                def ebody(e, _):
                    for j in range(D // 16):
                        sl = pl.ds(j * 16, 16)
                        t = 1.0 + jnp.exp(-r3[b][e, sl])
                        r2[b][e, sl] = r2[b][e, sl] / t
                    return 0
            lax.fori_loop(0, CH, ebody, 0)

        def sync_writes(b, k):
            base = base_of(k)
            if mode == "pass1":
                pltpu.sync_copy(r3[b], hat_h.at[pl.ds(base, CH)])
                pltpu.sync_copy(r1[b], acc.at[dstv[b]], add=True)
            else:
                pltpu.sync_copy(r2[b], acc.at[dstv[b]], add=True)

        def stage(b, k):
            nb = 1 - b
            load_idx(nb, k + 1)
            cps = issue_gathers(nb, k + 1)
            compute(b)
            sync_writes(b, k)
            for cp in cps:
                cp.wait()

        load_idx(0, 0)
        for cp in issue_gathers(0, 0):
            cp.wait()

        def gbody(g, _):
            stage(0, g * 2)
            stage(1, g * 2 + 1)
            return 0

        lax.fori_loop(0, cnt // 2, gbody, 0)

        @pl.when(cnt % 2 == 1)
        def _():
            stage(0, cnt - 1)

        plsc.subcore_barrier()

        @pl.when(c == 0)
        def _():
            _acc_out(acc, outa_h, s, n)

        @pl.when(c == 1)
        def _():
            _acc_out(acc, outb_h, s, n)

    return body


def _sc_pass1(n, E):
    return _sc_edge_pass(n, E, "pass1")


def _sc_pass2_hp(n, E):
    return _sc_edge_pass(n, E, "pass2_hp")


def _sc_pass2_h(n, E):
    return _sc_edge_pass(n, E, "pass2_h")


def _split_w(w):
    return jnp.transpose(w[:, :D]), jnp.transpose(w[:, D:])


def _gated_layer(lp, src, dst, n, h, p, b3e, with_p):
    wb1 = jnp.transpose(lp["B1"]["W"])
    wb2 = jnp.transpose(lp["B2"]["W"])
    wa2h, wa2p = _split_w(lp["A2"]["W"])
    E = src.shape[0]
    if with_p:
        wc2 = jnp.transpose(lp["C2"]["W"])
        b1h, b2h, vij, c2p = _nprep(n, True)(
            h, p, wb1, lp["B1"]["b"].reshape(1, D), wb2,
            lp["B2"]["b"].reshape(1, D), wa2h, wa2p,
            lp["A2"]["b"].reshape(1, D), wc2, lp["C2"]["b"].reshape(1, D))
    else:
        b1h, b2h, vij = _nprep(n, False)(
            h, p, wb1, lp["B1"]["b"].reshape(1, D), wb2,
            lp["B2"]["b"].reshape(1, D), wa2h, wa2p,
            lp["A2"]["b"].reshape(1, D))

    hat, ss0, ss1 = _sc_pass1(n, E)(src, dst, b1h, b2h, b3e)

    wa1h, wa1p = _split_w(lp["A1"]["W"])
    if with_p:
        h_agg, p_agg = _sc_pass2_hp(n, E)(
            src, dst, hat, jnp.concatenate([vij, c2p], axis=0))
        h_out, p_out = _nfin(n, True)(
            h, p, h_agg, p_agg, ss0, ss1,
            wa1h, wa1p, lp["A1"]["b"].reshape(1, D),
            jnp.transpose(lp["C1"]["W"]), lp["C1"]["b"].reshape(1, D),
            lp["bn_h_g"].reshape(1, D), lp["bn_h_b"].reshape(1, D))
        return h_out, p_out, hat
    ha0, ha1 = _sc_pass2_h(n, E)(src, dst, hat, vij)
    (h_out,) = _nfin(n, False)(
        h, p, ha0, ha1, ss0, ss1, wa1h, wa1p, lp["A1"]["b"].reshape(1, D),
        lp["bn_h_g"].reshape(1, D), lp["bn_h_b"].reshape(1, D))
    return h_out, None, hat


def _edge_in(lp, E, e_prev, hat_prev, st_prev):
    wb3 = jnp.transpose(lp["B3"]["W"])
    bb3 = lp["B3"]["b"].reshape(1, D)
    if hat_prev is None:
        b3e = _mm_bias(E, D, False)(e_prev, wb3, bb3)
        return e_prev, b3e
    e_out, b3e = _eprep(E)(e_prev, hat_prev, st_prev, wb3, bb3,
                           lp["bn_e_g"].reshape(1, D), lp["bn_e_b"].reshape(1, D))
    return e_out, b3e


def kernel(xp, ep_feats, pp_pos, edge_index_p, xl, el_feats, pl_pos,
           edge_index_l, ec_feats, edge_index_c, params):
    enc = params["enc"]
    n_p = xp.shape[0]
    n_l = xl.shape[0]
    n_c = n_p + n_l

    hp = _lin_tc(enc["p_node"], xp, ln=params["ln_p"])
    hl = _lin_tc(enc["l_node"], xl, ln=params["ln_l"])
    pp = _lin_tc(enc["p_pose"], pp_pos)
    plv = _lin_tc(enc["l_pose"], pl_pos)
    ep = _lin_tc(enc["p_edge"], ep_feats)
    el = _lin_tc(enc["l_edge"], el_feats)
    ec = _lin_tc(enc["c_edge"], ec_feats)

    sp, dp = edge_index_p[0], edge_index_p[1]
    sl, dl = edge_index_l[0], edge_index_l[1]
    sc, dc = edge_index_c[0], edge_index_c[1]
    E_p, E_l, E_c = sp.shape[0], sl.shape[0], sc.shape[0]

    hat_p = hat_l = hat_c = None
    st_p = st_l = st_c = None
    hc = None

    for i in range(3):
        lp_p = params["prot"][i]
        lp_l = params["lig"][i]
        lp_c = params["comp"][i]

        ep, b3e_p = _edge_in(lp_p, E_p, ep, hat_p, st_p)
        hp, pp, hat_p = _gated_layer(lp_p, sp, dp, n_p, hp, pp, b3e_p, True)

        el, b3e_l = _edge_in(lp_l, E_l, el, hat_l, st_l)
        hl, plv, hat_l = _gated_layer(lp_l, sl, dl, n_l, hl, plv, b3e_l, True)

        hc_in = jnp.concatenate([hp, hl], axis=0)
        pc_in = jnp.concatenate([pp, plv], axis=0)
        ec, b3e_c = _edge_in(lp_c, E_c, ec, hat_c, st_c)
        hc, _, hat_c = _gated_layer(lp_c, sc, dc, n_c, hc_in, pc_in,
                                    b3e_c, False)
        hp = hc[:n_p]
        hl = hc[n_p:]

        if i < 2:
            st_p = _estats(E_p)(hat_p)
            st_l = _estats(E_l)(hat_l)
            st_c = _estats(E_c)(hat_c)

    return (hp, hl, hc)


# speedup vs baseline: 3.3072x; 1.1193x over previous
import functools

import jax
import jax.numpy as jnp
from jax import lax
from jax.experimental import pallas as pl
from jax.experimental.pallas import tpu as pltpu
from jax.experimental.pallas import tpu_sc as plsc

F32 = jnp.float32
D = 128
BLK = 2000


def _pick_ch(n, E, nbuf):
    for ch in (128, 64, 32):
        if E % ch == 0 and 16 * (nbuf * ch * D + 4 * ch) + n * D <= 2_075_000:
            return ch
    return 16

_pallas_call = pl.pallas_call


@functools.lru_cache(None)
def _mm_bias(N, K, ln):
    grid = N // BLK

    def kern(*refs):
        if ln:
            x_ref, w_ref, b_ref, g_ref, gb_ref, o_ref = refs
        else:
            x_ref, w_ref, b_ref, o_ref = refs
        acc = jnp.dot(x_ref[...], w_ref[...], preferred_element_type=F32)
        acc = acc + b_ref[...]
        if ln:
            mu = jnp.mean(acc, axis=-1, keepdims=True)
            var = jnp.mean((acc - mu) ** 2, axis=-1, keepdims=True)
            acc = (acc - mu) / jnp.sqrt(var + 1e-5) * g_ref[...] + gb_ref[...]
        o_ref[...] = acc

    in_specs = [
        pl.BlockSpec((BLK, K), lambda i: (i, 0)),
        pl.BlockSpec((K, D), lambda i: (0, 0)),
        pl.BlockSpec((1, D), lambda i: (0, 0)),
    ]
    if ln:
        in_specs += [pl.BlockSpec((1, D), lambda i: (0, 0)),
                     pl.BlockSpec((1, D), lambda i: (0, 0))]
    return _pallas_call(
        kern, grid=(grid,), in_specs=in_specs,
        out_specs=pl.BlockSpec((BLK, D), lambda i: (i, 0)),
        out_shape=jax.ShapeDtypeStruct((N, D), F32))


def _lin_tc(p, x, ln=None):
    N, K = x.shape
    wt = jnp.transpose(p["W"])
    b = p["b"].reshape(1, D)
    if ln is None:
        return _mm_bias(N, K, False)(x, wt, b)
    return _mm_bias(N, K, True)(x, wt, b, ln["g"].reshape(1, D),
                                ln["b"].reshape(1, D))


@functools.lru_cache(None)
def _nprep(N, with_c2p):
    grid = N // BLK

    def kern(*refs):
        if with_c2p:
            (h_ref, p_ref, wb1, bb1, wb2, bb2, wa2h, wa2p, ba2, wc2, bc2,
             o1, o2, o3, o4) = refs
        else:
            h_ref, p_ref, wb1, bb1, wb2, bb2, wa2h, wa2p, ba2, o1, o2, o3 = refs
        h = h_ref[...]
        p = p_ref[...]
        o1[...] = jnp.dot(h, wb1[...], preferred_element_type=F32) + bb1[...]
        o2[...] = jnp.dot(h, wb2[...], preferred_element_type=F32) + bb2[...]
        o3[...] = (jnp.dot(h, wa2h[...], preferred_element_type=F32)
                   + jnp.dot(p, wa2p[...], preferred_element_type=F32)
                   + ba2[...])
        if with_c2p:
            o4[...] = jnp.dot(p, wc2[...], preferred_element_type=F32) + bc2[...]

    mat = pl.BlockSpec((D, D), lambda i: (0, 0))
    vec = pl.BlockSpec((1, D), lambda i: (0, 0))
    row = pl.BlockSpec((BLK, D), lambda i: (i, 0))
    if with_c2p:
        in_specs = [row, row, mat, vec, mat, vec, mat, mat, vec, mat, vec]
        n_out = 4
    else:
        in_specs = [row, row, mat, vec, mat, vec, mat, mat, vec]
        n_out = 3
    return _pallas_call(
        kern, grid=(grid,), in_specs=in_specs,
        out_specs=[row] * n_out,
        out_shape=[jax.ShapeDtypeStruct((N, D), F32)] * n_out)


@functools.lru_cache(None)
def _nfin(N, with_p):
    def kern(*refs):
        if with_p:
            (h_ref, p_ref, ha_ref, pa_ref, s0_ref, s1_ref,
             wa1h, wa1p, ba1, wc1, bc1, g_ref, gb_ref, ho_ref, po_ref) = refs
        else:
            (h_ref, p_ref, ha_ref, hb_ref, s0_ref, s1_ref,
             wa1h, wa1p, ba1, g_ref, gb_ref, ho_ref) = refs
        h = h_ref[...]
        p = p_ref[...]
        inv = 1.0 / (s0_ref[...] + s1_ref[...] + 1e-6)
        if with_p:
            h_agg = ha_ref[...] * inv
        else:
            h_agg = (ha_ref[...] + hb_ref[...]) * inv
        h_new = (jnp.dot(h, wa1h[...], preferred_element_type=F32)
                 + jnp.dot(p, wa1p[...], preferred_element_type=F32)
                 + ba1[...] + h_agg)
        mu = jnp.mean(h_new, axis=0, keepdims=True)
        var = jnp.mean((h_new - mu) ** 2, axis=0, keepdims=True)
        hb = (h_new - mu) / jnp.sqrt(var + 1e-5) * g_ref[...] + gb_ref[...]
        ho_ref[...] = h + jnp.maximum(hb, 0.0)
        if with_p:
            p_new = (jnp.dot(p, wc1[...], preferred_element_type=F32)
                     + bc1[...] + pa_ref[...] * inv)
            po_ref[...] = p + jnp.tanh(p_new)

    n_out = 2 if with_p else 1
    return _pallas_call(
        kern,
        out_shape=[jax.ShapeDtypeStruct((N, D), F32)] * n_out)


@functools.lru_cache(None)
def _estats(E):
    grid = E // BLK

    def kern(x_ref, o_ref):
        @pl.when(pl.program_id(0) == 0)
        def _():
            o_ref[...] = jnp.zeros((8, D), F32)
        x = x_ref[...]
        o_ref[0:1, :] += jnp.sum(x, axis=0, keepdims=True)
        o_ref[1:2, :] += jnp.sum(x * x, axis=0, keepdims=True)

    return _pallas_call(
        kern, grid=(grid,),
        in_specs=[pl.BlockSpec((BLK, D), lambda i: (i, 0))],
        out_specs=pl.BlockSpec((8, D), lambda i: (0, 0)),
        out_shape=jax.ShapeDtypeStruct((8, D), F32))


@functools.lru_cache(None)
def _eprep(E):
    grid = E // BLK
    inv = 1.0 / E

    def kern(e_ref, hat_ref, st_ref, wb3, bb3, g_ref, gb_ref, eo_ref, b3_ref):
        mu = st_ref[0:1, :] * inv
        var = st_ref[1:2, :] * inv - mu * mu
        hb = (hat_ref[...] - mu) / jnp.sqrt(var + 1e-5) * g_ref[...] + gb_ref[...]
        eo = e_ref[...] + jnp.maximum(hb, 0.0)
        eo_ref[...] = eo
        b3_ref[...] = jnp.dot(eo, wb3[...], preferred_element_type=F32) + bb3[...]

    row = pl.BlockSpec((BLK, D), lambda i: (i, 0))
    return _pallas_call(
        kern, grid=(grid,),
        in_specs=[row, row,
                  pl.BlockSpec((8, D), lambda i: (0, 0)),
                  pl.BlockSpec((D, D), lambda i: (0, 0)),
                  pl.BlockSpec((1, D), lambda i: (0, 0)),
                  pl.BlockSpec((1, D), lambda i: (0, 0)),
                  pl.BlockSpec((1, D), lambda i: (0, 0))],
        out_specs=[row, row],
        out_shape=[jax.ShapeDtypeStruct((E, D), F32)] * 2)


_MESH = dict(core_axis_name="c", subcore_axis_name="s")
_NC, _NS = 2, 16


def _zero_rows(r1):
    for j in range(8):
        for k8 in range(D // 16):
            r1[j, pl.ds(k8 * 16, 16)] = jnp.zeros((16,), F32)


def _acc_init(r1, acc, s, n):
    _zero_rows(r1)
    nz = n // 8
    cnt = (nz - s + _NS - 1) // _NS

    def zbody(k, _):
        r0 = (s + k * _NS) * 8
        pltpu.sync_copy(r1.at[pl.ds(0, 8)], acc.at[pl.ds(r0, 8)])
        return 0

    lax.fori_loop(0, cnt, zbody, 0)


def _acc_out(acc, out_ref, s, n):
    nz = n // 8
    cnt = (nz - s + _NS - 1) // _NS

    def obody(k, _):
        r0 = (s + k * _NS) * 8
        pltpu.sync_copy(acc.at[pl.ds(r0, 8)], out_ref.at[pl.ds(r0, 8)])
        return 0

    lax.fori_loop(0, cnt, obody, 0)


@functools.lru_cache(None)
def _sc_edge_pass(n, E, mode):
    nbuf = 6 if mode == "pass1" else 4
    CH = _pick_ch(n, E, nbuf)
    n_chunks = E // CH
    stride = _NS if mode == "pass2_hp" else _NC * _NS
    mesh = plsc.VectorSubcoreMesh(**_MESH)

    n_out = 3 if mode == "pass1" else 2
    out_type = tuple([jax.ShapeDtypeStruct((E, D), F32)] * (n_out - 2)
                     + [jax.ShapeDtypeStruct((n, D), F32)] * 2)
    scratch = ([pltpu.VMEM((CH,), jnp.int32)] * 4
               + [pltpu.VMEM((CH, D), F32)] * nbuf
               + [pltpu.VMEM_SHARED((n, D), F32)]
               + [pltpu.SemaphoreType.DMA] * 2)

    @functools.partial(pl.kernel, mesh=mesh, out_type=out_type,
                       scratch_types=scratch)
    def body(*refs):
        if mode == "pass1":
            (src_h, dst_h, b1_h, b2_h, b3_h, hat_h, outa_h, outb_h,
             sv0, sv1, dv0, dv1, r10, r11, r20, r21, r30, r31,
             acc, sg0, sg1) = refs
            r1 = (r10, r11)
            r2 = (r20, r21)
            r3 = (r30, r31)
        else:
            (src_h, dst_h, hat_h, tbl_h, outa_h, outb_h,
             sv0, sv1, dv0, dv1, r20, r21, r30, r31,
             acc, sg0, sg1) = refs
            r1 = (r20, r21)
            r2 = (r20, r21)
            r3 = (r30, r31)
        srcv = (sv0, sv1)
        dstv = (dv0, dv1)
        s_g = (sg0, sg1)

        c = lax.axis_index("c")
        s = lax.axis_index("s")
        wid = s if mode == "pass2_hp" else s * _NC + c

        _acc_init(r3[0], acc, s, n)
        plsc.subcore_barrier()

        cnt = (n_chunks - wid + stride - 1) // stride

        def base_of(k):
            return jnp.minimum(wid + k * stride, n_chunks - 1) * CH

        def load_idx(b, k):
            base = base_of(k)
            pltpu.sync_copy(src_h.at[pl.ds(base, CH)], srcv[b])
            pltpu.sync_copy(dst_h.at[pl.ds(base, CH)], dstv[b])
            if mode == "pass2_hp":
                off = c * n
                for j in range(CH // 16):
                    sl = pl.ds(j * 16, 16)
                    srcv[b][sl] = srcv[b][sl] + off

        def issue_gathers(b, k):
            base = base_of(k)
            if mode == "pass1":
                return (pltpu.async_copy(b1_h.at[srcv[b]], r1[b], s_g[b]),
                        pltpu.async_copy(b2_h.at[dstv[b]], r2[b], s_g[b]),
                        pltpu.async_copy(b3_h.at[pl.ds(base, CH)], r3[b], s_g[b]))
            return (pltpu.async_copy(tbl_h.at[srcv[b]], r2[b], s_g[b]),
                    pltpu.async_copy(hat_h.at[pl.ds(base, CH)], r3[b], s_g[b]))

        def compute(b):
            if mode == "pass1":
                def ebody(e, _):
                    for j in range(D // 16):
                        sl = pl.ds(j * 16, 16)
                        hat = r1[b][e, sl] + r2[b][e, sl] + r3[b][e, sl]
                        r3[b][e, sl] = hat
                        r1[b][e, sl] = 1.0 / (1.0 + jnp.exp(-hat))
                    return 0
            else:
                def ebody(e, _):
                    for j in range(D // 16):
                        sl = pl.ds(j * 16, 16)
                        t = 1.0 + jnp.exp(-r3[b][e, sl])
                        r2[b][e, sl] = r2[b][e, sl] / t
                    return 0
            lax.fori_loop(0, CH, ebody, 0)

        def sync_writes(b, k):
            base = base_of(k)
            if mode == "pass1":
                pltpu.sync_copy(r3[b], hat_h.at[pl.ds(base, CH)])
                pltpu.sync_copy(r1[b], acc.at[dstv[b]], add=True)
            else:
                pltpu.sync_copy(r2[b], acc.at[dstv[b]], add=True)

        def stage(b, k):
            nb = 1 - b
            load_idx(nb, k + 1)
            cps = issue_gathers(nb, k + 1)
            compute(b)
            sync_writes(b, k)
            for cp in cps:
                cp.wait()

        load_idx(0, 0)
        for cp in issue_gathers(0, 0):
            cp.wait()

        def gbody(g, _):
            stage(0, g * 2)
            stage(1, g * 2 + 1)
            return 0

        lax.fori_loop(0, cnt // 2, gbody, 0)

        @pl.when(cnt % 2 == 1)
        def _():
            stage(0, cnt - 1)

        plsc.subcore_barrier()

        @pl.when(c == 0)
        def _():
            _acc_out(acc, outa_h, s, n)

        @pl.when(c == 1)
        def _():
            _acc_out(acc, outb_h, s, n)

    return body


def _sc_pass1(n, E):
    return _sc_edge_pass(n, E, "pass1")


def _sc_pass2_hp(n, E):
    return _sc_edge_pass(n, E, "pass2_hp")


def _sc_pass2_h(n, E):
    return _sc_edge_pass(n, E, "pass2_h")


def _split_w(w):
    return jnp.transpose(w[:, :D]), jnp.transpose(w[:, D:])


def _gated_layer(lp, src, dst, n, h, p, b3e, with_p):
    wb1 = jnp.transpose(lp["B1"]["W"])
    wb2 = jnp.transpose(lp["B2"]["W"])
    wa2h, wa2p = _split_w(lp["A2"]["W"])
    E = src.shape[0]
    if with_p:
        wc2 = jnp.transpose(lp["C2"]["W"])
        b1h, b2h, vij, c2p = _nprep(n, True)(
            h, p, wb1, lp["B1"]["b"].reshape(1, D), wb2,
            lp["B2"]["b"].reshape(1, D), wa2h, wa2p,
            lp["A2"]["b"].reshape(1, D), wc2, lp["C2"]["b"].reshape(1, D))
    else:
        b1h, b2h, vij = _nprep(n, False)(
            h, p, wb1, lp["B1"]["b"].reshape(1, D), wb2,
            lp["B2"]["b"].reshape(1, D), wa2h, wa2p,
            lp["A2"]["b"].reshape(1, D))

    hat, ss0, ss1 = _sc_pass1(n, E)(src, dst, b1h, b2h, b3e)

    wa1h, wa1p = _split_w(lp["A1"]["W"])
    if with_p:
        h_agg, p_agg = _sc_pass2_hp(n, E)(
            src, dst, hat, jnp.concatenate([vij, c2p], axis=0))
        h_out, p_out = _nfin(n, True)(
            h, p, h_agg, p_agg, ss0, ss1,
            wa1h, wa1p, lp["A1"]["b"].reshape(1, D),
            jnp.transpose(lp["C1"]["W"]), lp["C1"]["b"].reshape(1, D),
            lp["bn_h_g"].reshape(1, D), lp["bn_h_b"].reshape(1, D))
        return h_out, p_out, hat
    ha0, ha1 = _sc_pass2_h(n, E)(src, dst, hat, vij)
    (h_out,) = _nfin(n, False)(
        h, p, ha0, ha1, ss0, ss1, wa1h, wa1p, lp["A1"]["b"].reshape(1, D),
        lp["bn_h_g"].reshape(1, D), lp["bn_h_b"].reshape(1, D))
    return h_out, None, hat


def _edge_in(lp, E, e_prev, hat_prev, st_prev):
    wb3 = jnp.transpose(lp["B3"]["W"])
    bb3 = lp["B3"]["b"].reshape(1, D)
    if hat_prev is None:
        b3e = _mm_bias(E, D, False)(e_prev, wb3, bb3)
        return e_prev, b3e
    e_out, b3e = _eprep(E)(e_prev, hat_prev, st_prev, wb3, bb3,
                           lp["bn_e_g"].reshape(1, D), lp["bn_e_b"].reshape(1, D))
    return e_out, b3e


def kernel(xp, ep_feats, pp_pos, edge_index_p, xl, el_feats, pl_pos,
           edge_index_l, ec_feats, edge_index_c, params):
    enc = params["enc"]
    n_p = xp.shape[0]
    n_l = xl.shape[0]
    n_c = n_p + n_l

    hp = _lin_tc(enc["p_node"], xp, ln=params["ln_p"])
    hl = _lin_tc(enc["l_node"], xl, ln=params["ln_l"])
    pp = _lin_tc(enc["p_pose"], pp_pos)
    plv = _lin_tc(enc["l_pose"], pl_pos)
    ep = _lin_tc(enc["p_edge"], ep_feats)
    el = _lin_tc(enc["l_edge"], el_feats)
    ec = _lin_tc(enc["c_edge"], ec_feats)

    sp, dp = edge_index_p[0], edge_index_p[1]
    sl, dl = edge_index_l[0], edge_index_l[1]
    sc, dc = edge_index_c[0], edge_index_c[1]
    E_p, E_l, E_c = sp.shape[0], sl.shape[0], sc.shape[0]

    hat_p = hat_l = hat_c = None
    st_p = st_l = st_c = None
    hc = None

    for i in range(3):
        lp_p = params["prot"][i]
        lp_l = params["lig"][i]
        lp_c = params["comp"][i]

        ep, b3e_p = _edge_in(lp_p, E_p, ep, hat_p, st_p)
        hp, pp, hat_p = _gated_layer(lp_p, sp, dp, n_p, hp, pp, b3e_p, True)

        el, b3e_l = _edge_in(lp_l, E_l, el, hat_l, st_l)
        hl, plv, hat_l = _gated_layer(lp_l, sl, dl, n_l, hl, plv, b3e_l, True)

        hc_in = jnp.concatenate([hp, hl], axis=0)
        pc_in = jnp.concatenate([pp, plv], axis=0)
        ec, b3e_c = _edge_in(lp_c, E_c, ec, hat_c, st_c)
        hc, _, hat_c = _gated_layer(lp_c, sc, dc, n_c, hc_in, pc_in,
                                    b3e_c, False)
        hp = hc[:n_p]
        hl = hc[n_p:]

        if i < 2:
            st_p = _estats(E_p)(hat_p)
            st_l = _estats(E_l)(hat_l)
            st_c = _estats(E_c)(hat_c)

    return (hp, hl, hc)
